# baseline (speedup 1.0000x reference)
import functools

import jax
import jax.numpy as jnp
from jax import lax
from jax.experimental import pallas as pl
from jax.experimental.pallas import tpu as pltpu
from jax.experimental.pallas import tpu_sc as plsc

BATCH = 16384
DIM = 32
NROWS = 1000000

_INFO = plsc.get_sparse_core_info()
_NC = _INFO.num_cores
_NS = _INFO.num_subcores
_NW = _NC * _NS
_BPW = BATCH // _NW
_CHUNK = 128
_NCHUNK = _BPW // _CHUNK

_TPK = NROWS // 128
_PITCH = _TPK + 1
_KWORDS = _PITCH * 1024
_NTILES = 4 * _PITCH
_CPW = _TPK // (_NW // 8)


def _detile_body(uT3, iT3, u_tail, i_tail, u_flat, i_flat, sem_w):
    wid = lax.axis_index("s") * _NC + lax.axis_index("c")
    unit = wid // 4
    quarter = wid % 4
    tbl = unit // 4
    k = unit % 4
    c0 = quarter * _CPW

    for t, (src3, tail, dst) in enumerate(((uT3, u_tail, u_flat),
                                           (iT3, i_tail, i_flat))):
        @pl.when(tbl == t)
        def _(src3=src3, tail=tail, dst=dst):
            def move(c, carry):
                pltpu.async_copy(
                    src3.at[k, :, pl.ds(pl.multiple_of((c0 + c) * 128, 128),
                                        128)],
                    dst.at[k * _PITCH + c0 + c], sem_w)
                return carry
            lax.fori_loop(0, _CPW, move, 0)
            @pl.when(quarter == 0)
            def _():
                pltpu.async_copy(tail.at[pl.ds(k * 8, 8), :],
                                 dst.at[k * _PITCH + _TPK], sem_w)
            region = dst.at[pl.ds(k * _PITCH + c0, _CPW)]
            pltpu.make_async_copy(region, region, sem_w).wait()
            @pl.when(quarter == 0)
            def _():
                tile = dst.at[k * _PITCH + _TPK]
                pltpu.make_async_copy(tile, tile, sem_w).wait()


def _gather_body(u_idx, p_idx, n_idx, u_flat, i_flat,
                 u_out, p_out, n_out,
                 ix_u, ix_p, ix_n, fbuf, ibuf,
                 rows_u, rows_p, rows_n,
                 sem_g, sem_s):
    wid = lax.axis_index("s") * _NC + lax.axis_index("c")
    cbase = wid * _NCHUNK

    pltpu.sync_copy(u_idx.at[pl.ds(cbase, _NCHUNK)], ix_u)
    pltpu.sync_copy(p_idx.at[pl.ds(cbase, _NCHUNK)], ix_p)
    pltpu.sync_copy(n_idx.at[pl.ds(cbase, _NCHUNK)], ix_n)

    for t, ix in enumerate((ix_u, ix_p, ix_n)):
        for jc in range(_NCHUNK):
            for v in range(_CHUNK // 16):
                iv = ix[jc, pl.ds(v * 16, 16)]
                fbuf[t, pl.ds(jc * _CHUNK + v * 16, 16)] = (
                    ((iv >> 7) << 10) | (iv & 127))

    tables = ((0, u_flat, rows_u), (1, i_flat, rows_p), (2, i_flat, rows_n))

    def per_d(d, carry):
        k = d // 8
        s = d % 8
        base = k * _KWORDS + s * 128

        for t in range(3):
            for c16 in range(_BPW // 16):
                ibuf[t * DIM + d, pl.ds(c16 * 16, 16)] = (
                    fbuf[t, pl.ds(c16 * 16, 16)] + base)

        for t, flat, rows in tables:
            for jc in range(_NCHUNK):
                pltpu.async_copy(
                    flat.at[ibuf.at[t * DIM + d, pl.ds(jc * _CHUNK, _CHUNK)]],
                    rows.at[k, s, pl.ds(jc * _CHUNK, _CHUNK)], sem_g)
        return carry
    lax.fori_loop(0, DIM, per_d, 0)

    for t, flat, rows in tables:
        for k in range(4):
            for s in range(8):
                pltpu.make_async_copy(flat.at[pl.ds(0, _BPW)],
                                      rows.at[k, s], sem_g).wait()

    stores = []
    for rows, out in ((rows_u, u_out), (rows_p, p_out), (rows_n, n_out)):
        for k in range(4):
            for c4 in range(_NCHUNK):
                stores.append(pltpu.async_copy(
                    rows.at[k, :, pl.ds(c4 * _CHUNK, _CHUNK)],
                    out.at[k, _NCHUNK * wid + c4], sem_s))
    for st in stores:
        st.wait()


@jax.jit
def _gather3(u_idx, p_idx, n_idx, uT3, iT3, u_tail, i_tail):
    flat_ty = jax.ShapeDtypeStruct((_NTILES, 8, 128), jnp.float32)
    detile = pl.kernel(
        _detile_body,
        mesh=plsc.VectorSubcoreMesh(core_axis_name="c", subcore_axis_name="s"),
        out_type=(flat_ty, flat_ty),
        scratch_types=[pltpu.SemaphoreType.DMA],
    )
    u_flat, i_flat = detile(uT3, iT3, u_tail, i_tail)

    out_ty = jax.ShapeDtypeStruct((4, BATCH // 128, 8, 128), jnp.float32)
    gather = pl.kernel(
        _gather_body,
        mesh=plsc.VectorSubcoreMesh(core_axis_name="c", subcore_axis_name="s"),
        compiler_params=pltpu.CompilerParams(use_tc_tiling_on_sc=False),
        out_type=(out_ty, out_ty, out_ty),
        scratch_types=[
            pltpu.VMEM((_NCHUNK, _CHUNK), jnp.int32),
            pltpu.VMEM((_NCHUNK, _CHUNK), jnp.int32),
            pltpu.VMEM((_NCHUNK, _CHUNK), jnp.int32),
            pltpu.VMEM((3, _BPW), jnp.int32),
            pltpu.VMEM((3 * DIM, _BPW), jnp.int32),
            pltpu.VMEM((4, 8, _BPW), jnp.float32),
            pltpu.VMEM((4, 8, _BPW), jnp.float32),
            pltpu.VMEM((4, 8, _BPW), jnp.float32),
            pltpu.SemaphoreType.DMA,
            pltpu.SemaphoreType.DMA,
        ],
    )
    u_flat1 = u_flat.reshape(_NTILES * 1024)
    i_flat1 = i_flat.reshape(_NTILES * 1024)
    return gather(u_idx, p_idx, n_idx, u_flat1, i_flat1)


def kernel(users, pos_items, neg_items, user_weight, item_weight):
    u = users.astype(jnp.int32).reshape(_NW * _NCHUNK, _CHUNK)
    p = pos_items.astype(jnp.int32).reshape(_NW * _NCHUNK, _CHUNK)
    n = neg_items.astype(jnp.int32).reshape(_NW * _NCHUNK, _CHUNK)
    uT3 = user_weight.T.reshape(4, 8, NROWS)
    iT3 = item_weight.T.reshape(4, 8, NROWS)

    def tail(w):
        return jnp.pad(w[NROWS - 64:].T, ((0, 0), (0, 64)))

    u4, p4, n4 = _gather3(u, p, n, uT3, iT3, tail(user_weight),
                          tail(item_weight))

    def unpack(x4):
        return x4.transpose(1, 3, 0, 2).reshape(BATCH, DIM)

    return (unpack(u4), unpack(p4), unpack(n4))


# speedup vs baseline: 8.4244x; 8.4244x over previous
import functools

import jax
import jax.numpy as jnp
from jax import lax
from jax.experimental import pallas as pl
from jax.experimental.pallas import tpu as pltpu
from jax.experimental.pallas import tpu_sc as plsc

BATCH = 16384
DIM = 32

_INFO = plsc.get_sparse_core_info()
_NC = _INFO.num_cores
_NS = _INFO.num_subcores
_NW = _NC * _NS
_BPW = BATCH // _NW
_CHUNK = 128
_NCHUNK = _BPW // _CHUNK


def _gather3_body(u_idx, p_idx, n_idx, uw, iw,
                  u_out, p_out, n_out,
                  idx_u, idx_p, idx_n,
                  rows_u, rows_p, rows_n,
                  sem_u, sem_p, sem_n, sem_s):
    wid = lax.axis_index("s") * _NC + lax.axis_index("c")
    base = wid * _BPW
    cbase = wid * _NCHUNK

    pltpu.sync_copy(u_idx.at[pl.ds(cbase, _NCHUNK)], idx_u)
    pltpu.sync_copy(p_idx.at[pl.ds(cbase, _NCHUNK)], idx_p)
    pltpu.sync_copy(n_idx.at[pl.ds(cbase, _NCHUNK)], idx_n)

    gathers = []
    for idx, table, rows, sem in ((idx_u, uw, rows_u, sem_u),
                                  (idx_p, iw, rows_p, sem_p),
                                  (idx_n, iw, rows_n, sem_n)):
        for j in range(_NCHUNK):
            gathers.append(
                pltpu.async_copy(table.at[idx.at[j]],
                                 rows.at[pl.ds(j * _CHUNK, _CHUNK)], sem))

    stores = []
    for g in range(3):
        for j in range(_NCHUNK):
            gathers[g * _NCHUNK + j].wait()
        rows, out = ((rows_u, u_out), (rows_p, p_out), (rows_n, n_out))[g]
        stores.append(
            pltpu.async_copy(rows, out.at[pl.ds(base, _BPW)], sem_s))
    for s in stores:
        s.wait()


@jax.jit
def _gather3(u_idx, p_idx, n_idx, uw, iw):
    out_ty = jax.ShapeDtypeStruct((BATCH, DIM), jnp.float32)
    run = pl.kernel(
        _gather3_body,
        mesh=plsc.VectorSubcoreMesh(core_axis_name="c", subcore_axis_name="s"),
        compiler_params=pltpu.CompilerParams(use_tc_tiling_on_sc=False),
        out_type=(out_ty, out_ty, out_ty),
        scratch_types=[
            pltpu.VMEM((_NCHUNK, _CHUNK), jnp.int32),
            pltpu.VMEM((_NCHUNK, _CHUNK), jnp.int32),
            pltpu.VMEM((_NCHUNK, _CHUNK), jnp.int32),
            pltpu.VMEM((_BPW, DIM), jnp.float32),
            pltpu.VMEM((_BPW, DIM), jnp.float32),
            pltpu.VMEM((_BPW, DIM), jnp.float32),
            pltpu.SemaphoreType.DMA,
            pltpu.SemaphoreType.DMA,
            pltpu.SemaphoreType.DMA,
            pltpu.SemaphoreType.DMA,
        ],
    )
    return run(u_idx, p_idx, n_idx, uw, iw)


def kernel(users, pos_items, neg_items, user_weight, item_weight):
    u = users.astype(jnp.int32).reshape(_NW * _NCHUNK, _CHUNK)
    p = pos_items.astype(jnp.int32).reshape(_NW * _NCHUNK, _CHUNK)
    n = neg_items.astype(jnp.int32).reshape(_NW * _NCHUNK, _CHUNK)
    return _gather3(u, p, n, user_weight, item_weight)
